# baseline (speedup 1.0000x reference)
import functools

import jax
import jax.numpy as jnp
from jax import lax
from jax.experimental import pallas as pl
from jax.experimental.pallas import tpu as pltpu
from jax.experimental.pallas import tpu_sc as plsc

E = 100000
N = 50000
D = 128
W = 0.1
NC, NS, L = 2, 16, 16
NPC = N // NC
RMAX = 4200
NPASS = -(-NPC // RMAX)
ESH = 6256
SEG = 2048
MB = SEG + 176
CH = 64
DV = D // L
RPT = -(-RMAX // (NS * 8)) * 8


def _sc_update_body(src, dst, tns, rp0, rp1, rp2, out,
                    e_src, e_dst, e_tw, m_scat, m_gath, m_w,
                    scat_a, gath_a, scat_b, gath_b,
                    rows_a0, rows_a1, rows_b0, rows_b1, tb16,
                    acc1, acc2, sem_ga, sem_gb, sem_sa, sem_sb):
  cid = lax.axis_index("c")
  sid = lax.axis_index("s")

  eoff = jnp.minimum(sid * ESH, E - ESH)
  overlap = sid * ESH - eoff
  pltpu.sync_copy(src.at[pl.ds(eoff, ESH)], e_src)
  pltpu.sync_copy(dst.at[pl.ds(eoff, ESH)], e_dst)
  pltpu.sync_copy(tns.at[pl.ds(eoff, ESH)], e_tw)
  pltpu.sync_copy(tns.at[pl.ds(E - L, L)], tb16)

  lane16 = lax.iota(jnp.int32, L)
  Tv = jnp.full((L,), tb16[...][L - 1], jnp.float32)
  decay_v = jnp.exp(-W * Tv)
  decay2_v = decay_v * decay_v

  neg1 = jnp.full((L,), -1, jnp.int32)
  zf = jnp.zeros((L,), jnp.float32)
  zi = jnp.zeros((L,), jnp.int32)
  for k in range(6):
    o = k * L

    @pl.when(o < overlap)
    def _():
      e_src[pl.ds(o, L)] = neg1
      e_dst[pl.ds(o, L)] = neg1

  def tw_body(k, carry):
    o = k * L
    e_tw[pl.ds(o, L)] = jnp.exp((e_tw[pl.ds(o, L)] - Tv) * W)
    return carry

  lax.fori_loop(0, ESH // L, tw_body, 0)

  trash_v = jnp.full((L,), MB - L, jnp.int32) + lane16
  sets = ((scat_a, gath_a, rows_a0, rows_a1, sem_ga, sem_sa),
          (scat_b, gath_b, rows_b0, rows_b1, sem_gb, sem_sb))

  def one_pass(p, carry):
    range_lo = cid * NPC + p * RMAX
    range_n = jnp.minimum(RMAX, NPC - p * RMAX)
    my_lo = sid * RPT

    def init_ck(i, carry2):
      cs = pl.multiple_of(jnp.minimum(my_lo + i * CH, range_n - CH), 8)
      gs = pl.multiple_of(range_lo + cs, 8)
      pltpu.sync_copy(rp1.at[pl.ds(gs, CH)], rows_a0)
      pltpu.sync_copy(rp2.at[pl.ds(gs, CH)], rows_a1)

      def init_row(j, carry3):
        for v in range(DV):
          slc = pl.ds(v * L, L)
          rows_a0[j, slc] = rows_a0[j, slc] * decay_v
          rows_a1[j, slc] = rows_a1[j, slc] * decay2_v
        return carry3

      lax.fori_loop(0, CH, init_row, 0)
      pltpu.sync_copy(rows_a0, acc1.at[pl.ds(cs, CH)])
      pltpu.sync_copy(rows_a1, acc2.at[pl.ds(cs, CH)])
      return carry2

    lax.fori_loop(0, -(-RPT // CH), init_ck, 0)

    plsc.subcore_barrier()

    def drain(cnt):
      for k in range(CH // L):
        pidx = jnp.full((L,), cnt + k * L, jnp.int32) + lane16
        plsc.store_scatter(m_scat, [pidx], zi)
        plsc.store_scatter(m_gath, [pidx], zi)
        plsc.store_scatter(m_w, [pidx], zf)

      nch = (cnt + CH - 1) // CH

      def pair(q, carry2):
        for half in range(2):
          c = q * 2 + half
          sc_b, ga_b, r0, r1, sg, ss = sets[half]

          @pl.when((c < nch) & (c >= 2))
          def _():
            pltpu.make_async_copy(r0, acc1.at[sc_b], ss).wait()
            pltpu.make_async_copy(r1, acc2.at[sc_b], ss).wait()

          @pl.when(c < nch)
          def _():
            for k in range(CH // L):
              sc_b[pl.ds(k * L, L)] = m_scat[pl.ds(c * CH + k * L, L)]
              ga_b[pl.ds(k * L, L)] = m_gath[pl.ds(c * CH + k * L, L)]
            pltpu.async_copy(rp0.at[ga_b], r0, sg)
            pltpu.async_copy(rp1.at[ga_b], r1, sg)

        for half in range(2):
          c = q * 2 + half
          sc_b, ga_b, r0, r1, sg, ss = sets[half]

          @pl.when(c < nch)
          def _():
            pltpu.make_async_copy(rp0.at[ga_b], r0, sg).wait()
            pltpu.make_async_copy(rp1.at[ga_b], r1, sg).wait()

            def srow(j, carry3):
              w1 = jnp.full((L,), m_w[pl.ds(c * CH + j, L)][0], jnp.float32)
              w2 = w1 * decay_v
              for v in range(DV):
                slc = pl.ds(v * L, L)
                r0[j, slc] = r0[j, slc] * w1
                r1[j, slc] = r1[j, slc] * w2
              return carry3

            lax.fori_loop(0, CH, srow, 0)
            pltpu.async_copy(r0, acc1.at[sc_b], ss, add=True)
            pltpu.async_copy(r1, acc2.at[sc_b], ss, add=True)

        return carry2

      lax.fori_loop(0, (nch + 1) // 2, pair, 0)
      for half in range(2):
        sc_b, ga_b, r0, r1, sg, ss = sets[half]

        @pl.when(nch >= half + 1)
        def _():
          pltpu.make_async_copy(r0, acc1.at[sc_b], ss).wait()
          pltpu.make_async_copy(r1, acc2.at[sc_b], ss).wait()

    lo_v = jnp.full((L,), range_lo, jnp.int32)
    hi_v = lo_v + range_n

    def flt(k, cnt):
      o = k * L
      s16 = e_src[pl.ds(o, L)]
      d16 = e_dst[pl.ds(o, L)]
      w16 = e_tw[pl.ds(o, L)]
      for tg, sc in ((s16, d16), (d16, s16)):
        m = (tg >= lo_v) & (tg < hi_v)
        cs = plsc.cumsum(jnp.where(m, 1, 0))
        idx = jnp.where(m, cs - 1 + cnt, trash_v)
        plsc.store_scatter(m_scat, [idx], tg - lo_v)
        plsc.store_scatter(m_gath, [idx], sc)
        plsc.store_scatter(m_w, [idx], w16)
        cnt = cnt + cs[L - 1]

      @pl.when(cnt >= SEG)
      def _():
        drain(cnt)

      return jnp.where(cnt >= SEG, jnp.int32(0), cnt)

    cnt = lax.fori_loop(0, ESH // L, flt, jnp.int32(0))
    drain(cnt)

    plsc.subcore_barrier()

    def wb_ck(i, carry2):
      cs = pl.multiple_of(jnp.minimum(my_lo + i * CH, range_n - CH), 8)
      gs = pl.multiple_of(range_lo + cs, 8)
      pltpu.sync_copy(acc1.at[pl.ds(cs, CH)], rows_a0)
      pltpu.sync_copy(rows_a0, out.at[0, pl.ds(gs, CH)])
      pltpu.sync_copy(acc2.at[pl.ds(cs, CH)], rows_a1)
      pltpu.sync_copy(rows_a1, out.at[1, pl.ds(gs, CH)])
      return carry2

    lax.fori_loop(0, -(-RPT // CH), wb_ck, 0)

    plsc.subcore_barrier()
    return carry

  lax.fori_loop(0, NPASS, one_pass, 0)


_rp_update = functools.partial(
    pl.kernel,
    out_type=jax.ShapeDtypeStruct((2, N, D), jnp.float32),
    compiler_params=pltpu.CompilerParams(
        use_tc_tiling_on_sc=False, needs_layout_passes=False),
    mesh=plsc.VectorSubcoreMesh(
        core_axis_name="c", subcore_axis_name="s",
        num_cores=NC, num_subcores=NS),
    scratch_types=[
        pltpu.VMEM((ESH,), jnp.int32),
        pltpu.VMEM((ESH,), jnp.int32),
        pltpu.VMEM((ESH,), jnp.float32),
        pltpu.VMEM((MB,), jnp.int32),
        pltpu.VMEM((MB,), jnp.int32),
        pltpu.VMEM((MB,), jnp.float32),
        pltpu.VMEM((CH,), jnp.int32),
        pltpu.VMEM((CH,), jnp.int32),
        pltpu.VMEM((CH,), jnp.int32),
        pltpu.VMEM((CH,), jnp.int32),
        pltpu.VMEM((CH, D), jnp.float32),
        pltpu.VMEM((CH, D), jnp.float32),
        pltpu.VMEM((CH, D), jnp.float32),
        pltpu.VMEM((CH, D), jnp.float32),
        pltpu.VMEM((L,), jnp.float32),
        pltpu.MemorySpace.VMEM_SHARED((RMAX, D), jnp.float32),
        pltpu.MemorySpace.VMEM_SHARED((RMAX, D), jnp.float32),
        pltpu.SemaphoreType.DMA,
        pltpu.SemaphoreType.DMA,
        pltpu.SemaphoreType.DMA,
        pltpu.SemaphoreType.DMA,
    ],
)(_sc_update_body)


def kernel(src_node_ids, dst_node_ids, node_interact_times, rp0, rp1, rp2):
  return _rp_update(
      src_node_ids.astype(jnp.int32),
      dst_node_ids.astype(jnp.int32),
      node_interact_times.astype(jnp.float32),
      rp0, rp1, rp2)


# speedup vs baseline: 1.0469x; 1.0469x over previous
import functools

import jax
import jax.numpy as jnp
from jax import lax
from jax.experimental import pallas as pl
from jax.experimental.pallas import tpu as pltpu
from jax.experimental.pallas import tpu_sc as plsc

E = 100000
N = 50000
D = 128
W = 0.1
NC, NS, L = 2, 16, 16
NPC = N // NC
RMAX = 4200
NPASS = -(-NPC // RMAX)
ESH = 6256
SEG = 2048
MB = SEG + 176
CH = 64
DV = D // L
RPT = -(-RMAX // (NS * 8)) * 8


def _sc_update_body(src, dst, tns, rp0, rp1, rp2, out,
                    e_src, e_dst, e_tw, m_scat, m_gath, m_w,
                    scat_a, gath_a, scat_b, gath_b,
                    rows_a0, rows_a1, rows_b0, rows_b1, tb16,
                    acc1, acc2, sem_ga, sem_gb, sem_sa, sem_sb):
  cid = lax.axis_index("c")
  sid = lax.axis_index("s")

  eoff = jnp.minimum(sid * ESH, E - ESH)
  overlap = sid * ESH - eoff
  pltpu.sync_copy(src.at[pl.ds(eoff, ESH)], e_src)
  pltpu.sync_copy(dst.at[pl.ds(eoff, ESH)], e_dst)
  pltpu.sync_copy(tns.at[pl.ds(eoff, ESH)], e_tw)
  pltpu.sync_copy(tns.at[pl.ds(E - L, L)], tb16)

  lane16 = lax.iota(jnp.int32, L)
  Tv = jnp.full((L,), tb16[...][L - 1], jnp.float32)
  decay_v = jnp.exp(-W * Tv)
  decay2_v = decay_v * decay_v

  neg1 = jnp.full((L,), -1, jnp.int32)
  zf = jnp.zeros((L,), jnp.float32)
  zi = jnp.zeros((L,), jnp.int32)
  for k in range(6):
    o = k * L

    @pl.when(o < overlap)
    def _():
      e_src[pl.ds(o, L)] = neg1
      e_dst[pl.ds(o, L)] = neg1

  def tw_body(k, carry):
    o = k * L
    e_tw[pl.ds(o, L)] = jnp.exp((e_tw[pl.ds(o, L)] - Tv) * W)
    return carry

  lax.fori_loop(0, ESH // L, tw_body, 0)

  trash_v = jnp.full((L,), MB - L, jnp.int32) + lane16
  sets = ((scat_a, gath_a, rows_a0, rows_a1, sem_ga, sem_sa),
          (scat_b, gath_b, rows_b0, rows_b1, sem_gb, sem_sb))

  def one_pass(p, carry):
    range_lo = cid * NPC + p * RMAX
    range_n = jnp.minimum(RMAX, NPC - p * RMAX)
    my_lo = sid * RPT

    def init_ck(i, carry2):
      cs = pl.multiple_of(jnp.minimum(my_lo + i * CH, range_n - CH), 8)
      gs = pl.multiple_of(range_lo + cs, 8)
      ra = pltpu.async_copy(rp1.at[pl.ds(gs, CH)], rows_a0, sem_ga)
      rb = pltpu.async_copy(rp2.at[pl.ds(gs, CH)], rows_a1, sem_ga)
      ra.wait()
      rb.wait()

      def init_row(j, carry3):
        for v in range(DV):
          slc = pl.ds(v * L, L)
          rows_a0[j, slc] = rows_a0[j, slc] * decay_v
          rows_a1[j, slc] = rows_a1[j, slc] * decay2_v
        return carry3

      lax.fori_loop(0, CH, init_row, 0)
      wa = pltpu.async_copy(rows_a0, acc1.at[pl.ds(cs, CH)], sem_sa)
      wb = pltpu.async_copy(rows_a1, acc2.at[pl.ds(cs, CH)], sem_sa)
      wa.wait()
      wb.wait()
      return carry2

    lax.fori_loop(0, -(-RPT // CH), init_ck, 0)

    plsc.subcore_barrier()

    def drain(cnt):
      for k in range(CH // L):
        pidx = jnp.full((L,), cnt + k * L, jnp.int32) + lane16
        plsc.store_scatter(m_scat, [pidx], zi)
        plsc.store_scatter(m_gath, [pidx], zi)
        plsc.store_scatter(m_w, [pidx], zf)

      nch = (cnt + CH - 1) // CH

      def pair(q, carry2):
        for half in range(2):
          c = q * 2 + half
          sc_b, ga_b, r0, r1, sg, ss = sets[half]

          @pl.when((c < nch) & (c >= 2))
          def _():
            pltpu.make_async_copy(r0, acc1.at[sc_b], ss).wait()
            pltpu.make_async_copy(r1, acc2.at[sc_b], ss).wait()

          @pl.when(c < nch)
          def _():
            for k in range(CH // L):
              sc_b[pl.ds(k * L, L)] = m_scat[pl.ds(c * CH + k * L, L)]
              ga_b[pl.ds(k * L, L)] = m_gath[pl.ds(c * CH + k * L, L)]
            pltpu.async_copy(rp0.at[ga_b], r0, sg)
            pltpu.async_copy(rp1.at[ga_b], r1, sg)

        for half in range(2):
          c = q * 2 + half
          sc_b, ga_b, r0, r1, sg, ss = sets[half]

          @pl.when(c < nch)
          def _():
            pltpu.make_async_copy(rp0.at[ga_b], r0, sg).wait()
            pltpu.make_async_copy(rp1.at[ga_b], r1, sg).wait()

            def srow(j, carry3):
              w1 = jnp.full((L,), m_w[pl.ds(c * CH + j, L)][0], jnp.float32)
              w2 = w1 * decay_v
              for v in range(DV):
                slc = pl.ds(v * L, L)
                r0[j, slc] = r0[j, slc] * w1
                r1[j, slc] = r1[j, slc] * w2
              return carry3

            lax.fori_loop(0, CH, srow, 0)
            pltpu.async_copy(r0, acc1.at[sc_b], ss, add=True)
            pltpu.async_copy(r1, acc2.at[sc_b], ss, add=True)

        return carry2

      lax.fori_loop(0, (nch + 1) // 2, pair, 0)
      for half in range(2):
        sc_b, ga_b, r0, r1, sg, ss = sets[half]

        @pl.when(nch >= half + 1)
        def _():
          pltpu.make_async_copy(r0, acc1.at[sc_b], ss).wait()
          pltpu.make_async_copy(r1, acc2.at[sc_b], ss).wait()

    lo_v = jnp.full((L,), range_lo, jnp.int32)
    hi_v = lo_v + range_n

    def flt(k, cnt):
      o = k * L
      s16 = e_src[pl.ds(o, L)]
      d16 = e_dst[pl.ds(o, L)]
      w16 = e_tw[pl.ds(o, L)]
      for tg, sc in ((s16, d16), (d16, s16)):
        m = (tg >= lo_v) & (tg < hi_v)
        cs = plsc.cumsum(jnp.where(m, 1, 0))
        idx = jnp.where(m, cs - 1 + cnt, trash_v)
        plsc.store_scatter(m_scat, [idx], tg - lo_v)
        plsc.store_scatter(m_gath, [idx], sc)
        plsc.store_scatter(m_w, [idx], w16)
        cnt = cnt + cs[L - 1]

      @pl.when(cnt >= SEG)
      def _():
        drain(cnt)

      return jnp.where(cnt >= SEG, jnp.int32(0), cnt)

    cnt = lax.fori_loop(0, ESH // L, flt, jnp.int32(0))
    drain(cnt)

    plsc.subcore_barrier()

    wb_descs = []
    for i in range(-(-RPT // CH)):
      cs = pl.multiple_of(jnp.minimum(my_lo + i * CH, range_n - CH), 8)
      gs = pl.multiple_of(range_lo + cs, 8)
      wb_descs.append(
          pltpu.async_copy(acc1.at[pl.ds(cs, CH)], out.at[0, pl.ds(gs, CH)],
                           sem_sa))
      wb_descs.append(
          pltpu.async_copy(acc2.at[pl.ds(cs, CH)], out.at[1, pl.ds(gs, CH)],
                           sem_sb))
    for d in wb_descs:
      d.wait()

    plsc.subcore_barrier()
    return carry

  lax.fori_loop(0, NPASS, one_pass, 0)


_rp_update = functools.partial(
    pl.kernel,
    out_type=jax.ShapeDtypeStruct((2, N, D), jnp.float32),
    compiler_params=pltpu.CompilerParams(
        use_tc_tiling_on_sc=False, needs_layout_passes=False),
    mesh=plsc.VectorSubcoreMesh(
        core_axis_name="c", subcore_axis_name="s",
        num_cores=NC, num_subcores=NS),
    scratch_types=[
        pltpu.VMEM((ESH,), jnp.int32),
        pltpu.VMEM((ESH,), jnp.int32),
        pltpu.VMEM((ESH,), jnp.float32),
        pltpu.VMEM((MB,), jnp.int32),
        pltpu.VMEM((MB,), jnp.int32),
        pltpu.VMEM((MB,), jnp.float32),
        pltpu.VMEM((CH,), jnp.int32),
        pltpu.VMEM((CH,), jnp.int32),
        pltpu.VMEM((CH,), jnp.int32),
        pltpu.VMEM((CH,), jnp.int32),
        pltpu.VMEM((CH, D), jnp.float32),
        pltpu.VMEM((CH, D), jnp.float32),
        pltpu.VMEM((CH, D), jnp.float32),
        pltpu.VMEM((CH, D), jnp.float32),
        pltpu.VMEM((L,), jnp.float32),
        pltpu.MemorySpace.VMEM_SHARED((RMAX, D), jnp.float32),
        pltpu.MemorySpace.VMEM_SHARED((RMAX, D), jnp.float32),
        pltpu.SemaphoreType.DMA,
        pltpu.SemaphoreType.DMA,
        pltpu.SemaphoreType.DMA,
        pltpu.SemaphoreType.DMA,
    ],
)(_sc_update_body)


def kernel(src_node_ids, dst_node_ids, node_interact_times, rp0, rp1, rp2):
  return _rp_update(
      src_node_ids.astype(jnp.int32),
      dst_node_ids.astype(jnp.int32),
      node_interact_times.astype(jnp.float32),
      rp0, rp1, rp2)
